# Initial kernel scaffold; baseline (speedup 1.0000x reference)
import jax
import jax.numpy as jnp
from jax.experimental import pallas as pl


def kernel(x, univariate_vars, univariate_tables, bivariate_vars, bivariate_tables):
    raise NotImplementedError("write your pallas kernel here")


# speedup vs baseline: 1.4101x; 1.4101x over previous
import functools

import jax
import jax.numpy as jnp
from jax import lax
from jax.experimental import pallas as pl
from jax.experimental.pallas import tpu as pltpu
from jax.experimental.pallas import tpu_sc as plsc

B = 16
V = 50000
E = 800000
NC = 2
NS = 16
NW = NC * NS
ET_TOTAL = E + V
ET_PAD = 851968
EPT = ET_PAD // NW
K = 2048
NCHUNK = EPT // K
assert EPT == NCHUNK * K


def _sc_body(ev_hbm, et_hbm, xt_hbm, out_hbm,
             idx2, idx0b, idx1b, x0r, x1r, tblc, accb, sem0, sem1):
    wid = lax.axis_index("s") * NC + lax.axis_index("c")
    iota = lax.iota(jnp.int32, 16)

    accb[...] = jnp.zeros((16,), jnp.float32)

    @pl.loop(0, NCHUNK)
    def _chunk(i):
        base = wid * EPT + i * K
        pltpu.sync_copy(ev_hbm.at[pl.ds(2 * base, 2 * K)], idx2)

        @pl.loop(0, K // 16)
        def _deint(g):
            iv = (iota + g * 16) * 2
            idx0b[pl.ds(g * 16, 16)] = plsc.load_gather(idx2, [iv])
            idx1b[pl.ds(g * 16, 16)] = plsc.load_gather(idx2, [iv + 1])

        d0 = pltpu.async_copy(xt_hbm.at[idx0b], x0r, sem0)
        d1 = pltpu.async_copy(xt_hbm.at[idx1b], x1r, sem1)
        pltpu.sync_copy(et_hbm.at[pl.ds(4 * base, 4 * K)], tblc)
        d0.wait()
        d1.wait()

        acc0 = jnp.zeros((16,), jnp.float32)

        @pl.loop(0, K, init_carry=acc0, unroll=8)
        def _edge(e, acc):
            x0v = x0r[e]
            x1v = x1r[e]
            sel = (x0v.astype(jnp.int32) * 2 + x1v.astype(jnp.int32)) + e * 4
            w = plsc.load_gather(tblc, [sel])
            return acc + w

        accb[...] = accb[...] + _edge

    pltpu.sync_copy(accb, out_hbm.at[wid])


@functools.partial(
    pl.kernel,
    out_type=jax.ShapeDtypeStruct((NW, 16), jnp.float32),
    mesh=plsc.VectorSubcoreMesh(core_axis_name="c", subcore_axis_name="s"),
    compiler_params=pltpu.CompilerParams(
        needs_layout_passes=False, use_tc_tiling_on_sc=False),
    scratch_types=[
        pltpu.VMEM((2 * K,), jnp.int32),
        pltpu.VMEM((K,), jnp.int32),
        pltpu.VMEM((K,), jnp.int32),
        pltpu.VMEM((K, 16), jnp.float32),
        pltpu.VMEM((K, 16), jnp.float32),
        pltpu.VMEM((4 * K,), jnp.float32),
        pltpu.VMEM((16,), jnp.float32),
        pltpu.SemaphoreType.DMA,
        pltpu.SemaphoreType.DMA,
    ],
)
def _mn_edges(ev_hbm, et_hbm, xt_hbm, out_hbm, *scratch):
    _sc_body(ev_hbm, et_hbm, xt_hbm, out_hbm, *scratch)


def kernel(x, univariate_vars, univariate_tables, bivariate_vars, bivariate_tables):
    npad = ET_PAD - ET_TOTAL
    uv = univariate_vars.astype(jnp.int32)
    uni_vars2 = jnp.stack([uv, uv], axis=1)
    eye = jnp.eye(2, dtype=jnp.float32)
    uni_tbl4 = univariate_tables[:, :, None] * eye[None]
    ev = jnp.concatenate([
        bivariate_vars.astype(jnp.int32),
        uni_vars2,
        jnp.zeros((npad, 2), jnp.int32),
    ], axis=0).reshape(-1)
    et = jnp.concatenate([
        bivariate_tables.reshape(E, 4),
        uni_tbl4.reshape(V, 4),
        jnp.zeros((npad, 4), jnp.float32),
    ], axis=0).reshape(-1)
    xt = x.T.astype(jnp.float32)
    partials = _mn_edges(ev, et, xt)
    return jnp.sum(partials, axis=0)
